# baseline (speedup 1.0000x reference)
import jax
import jax.numpy as jnp
from jax.experimental import pallas as pl
from jax.experimental.pallas import tpu as pltpu

_B, _C, _H, _W = 8, 384, 200, 176
_O_PAD = 32
_HB = 8


def _head_kernel(x_ref, w_ref, b_ref, cls_ref, reg_ref, dir_ref):
    xm = x_ref[0].reshape(_HB * _W, _C)
    acc = jax.lax.dot_general(
        xm, w_ref[...],
        dimension_numbers=(((1,), (0,)), ((), ())),
        preferred_element_type=jnp.float32,
        precision=jax.lax.Precision.DEFAULT,
    ) + b_ref[...]
    acc3 = acc.reshape(_HB, _W, _O_PAD)
    t3 = jnp.transpose(acc3, (0, 2, 1))
    for h in range(_HB):
        t = t3[h]
        cls_ref[0, :, h, :] = t[0:2]
        reg_ref[0, :, h, :] = t[2:16]
        dir_ref[0, :, h, :] = t[16:20]


def kernel(x, W_cls, b_cls, W_reg, b_reg, W_dir, b_dir):
    w = jnp.concatenate([W_cls, W_reg, W_dir], axis=1)
    w = jnp.pad(w, ((0, 0), (0, _O_PAD - w.shape[1])))
    b = jnp.concatenate([b_cls, b_reg, b_dir])
    b = jnp.pad(b, (0, _O_PAD - b.shape[0]))[None, :]

    xt = jnp.transpose(x, (0, 2, 3, 1))
    n_blocks = _H // _HB

    cls_o, reg_o, dir_o = pl.pallas_call(
        _head_kernel,
        grid=(_B, n_blocks),
        in_specs=[
            pl.BlockSpec((1, _HB, _W, _C), lambda bi, hi: (bi, hi, 0, 0)),
            pl.BlockSpec((_C, _O_PAD), lambda bi, hi: (0, 0)),
            pl.BlockSpec((1, _O_PAD), lambda bi, hi: (0, 0)),
        ],
        out_specs=[
            pl.BlockSpec((1, 2, _HB, _W), lambda bi, hi: (bi, 0, hi, 0)),
            pl.BlockSpec((1, 14, _HB, _W), lambda bi, hi: (bi, 0, hi, 0)),
            pl.BlockSpec((1, 4, _HB, _W), lambda bi, hi: (bi, 0, hi, 0)),
        ],
        out_shape=[
            jax.ShapeDtypeStruct((_B, 2, _H, _W), jnp.float32),
            jax.ShapeDtypeStruct((_B, 14, _H, _W), jnp.float32),
            jax.ShapeDtypeStruct((_B, 4, _H, _W), jnp.float32),
        ],
        compiler_params=pltpu.CompilerParams(
            dimension_semantics=("parallel", "parallel"),
        ),
    )(xt, w, b)

    return (cls_o, reg_o, dir_o)


# speedup vs baseline: 1.5730x; 1.5730x over previous
import jax
import jax.numpy as jnp
from jax.experimental import pallas as pl
from jax.experimental.pallas import tpu as pltpu

_B, _C, _H, _W = 8, 384, 200, 176
_O_PAD = 32
_HB = 40


def _head_kernel(x_ref, w_ref, b_ref, cls_ref, reg_ref, dir_ref):
    xm = x_ref[0].reshape(_HB * _W, _C)
    acc = jax.lax.dot_general(
        xm, w_ref[...],
        dimension_numbers=(((1,), (0,)), ((), ())),
        preferred_element_type=jnp.float32,
        precision=jax.lax.Precision.DEFAULT,
    ) + b_ref[...]
    acc3 = acc.reshape(_HB, _W, _O_PAD)
    t3 = jnp.transpose(acc3, (0, 2, 1))
    for h in range(_HB):
        t = t3[h]
        cls_ref[0, :, h, :] = t[0:2]
        reg_ref[0, :, h, :] = t[2:16]
        dir_ref[0, :, h, :] = t[16:20]


def kernel(x, W_cls, b_cls, W_reg, b_reg, W_dir, b_dir):
    w = jnp.concatenate([W_cls, W_reg, W_dir], axis=1)
    w = jnp.pad(w, ((0, 0), (0, _O_PAD - w.shape[1])))
    b = jnp.concatenate([b_cls, b_reg, b_dir])
    b = jnp.pad(b, (0, _O_PAD - b.shape[0]))[None, :]

    xt = jnp.transpose(x, (0, 2, 3, 1))
    n_blocks = _H // _HB

    cls_o, reg_o, dir_o = pl.pallas_call(
        _head_kernel,
        grid=(_B, n_blocks),
        in_specs=[
            pl.BlockSpec((1, _HB, _W, _C), lambda bi, hi: (bi, hi, 0, 0)),
            pl.BlockSpec((_C, _O_PAD), lambda bi, hi: (0, 0)),
            pl.BlockSpec((1, _O_PAD), lambda bi, hi: (0, 0)),
        ],
        out_specs=[
            pl.BlockSpec((1, 2, _HB, _W), lambda bi, hi: (bi, 0, hi, 0)),
            pl.BlockSpec((1, 14, _HB, _W), lambda bi, hi: (bi, 0, hi, 0)),
            pl.BlockSpec((1, 4, _HB, _W), lambda bi, hi: (bi, 0, hi, 0)),
        ],
        out_shape=[
            jax.ShapeDtypeStruct((_B, 2, _H, _W), jnp.float32),
            jax.ShapeDtypeStruct((_B, 14, _H, _W), jnp.float32),
            jax.ShapeDtypeStruct((_B, 4, _H, _W), jnp.float32),
        ],
        compiler_params=pltpu.CompilerParams(
            dimension_semantics=("parallel", "parallel"),
        ),
    )(xt, w, b)

    return (cls_o, reg_o, dir_o)
